# baseline (speedup 1.0000x reference)
import functools

import jax
import jax.numpy as jnp
from jax import lax
from jax.experimental import pallas as pl
from jax.experimental.pallas import tpu as pltpu
from jax.experimental.pallas import tpu_sc as plsc

_CLS_W = 10.0
_REG_W = 8.0
_GAMMA = 2.0
_ALPHA = 0.25

_B, _T, _M, _FW, _D = 256, 128, 20, 8, 3
_BT = _B * _T
_ROW = _FW * _D
_POSE_LANES = _FW * 16

_R = 512
_G = _BT // _R
_RC = 2048
_GC = _BT // _RC

_NW = 32
_NPW = _BT // _NW
_CHUNK = 128
_NCH = _NPW // _CHUNK


def _hdot(a, b, dims=None):
    if dims is None:
        return lax.dot(a, b, precision=lax.Precision.HIGHEST)
    return lax.dot_general(a, b, (dims, ((), ())), precision=lax.Precision.HIGHEST)


def _tc_main_body(poses_ref, cls0_ref, cls1_ref, anchor_ref,
                  idx_ref, t24_ref, f0_ref, f1_ref):
    i = pl.program_id(0)
    poses = poses_ref[...]
    anchor = anchor_ref[...]

    r = lax.broadcasted_iota(jnp.int32, (_POSE_LANES, 2 * _FW), 0)
    c = lax.broadcasted_iota(jnp.int32, (_POSE_LANES, 2 * _FW), 1)
    sxy = ((c // 2) * 16 + 3 + (c % 2) * 4 == r).astype(jnp.float32)
    r3 = lax.broadcasted_iota(jnp.int32, (_POSE_LANES, _ROW), 0)
    c3 = lax.broadcasted_iota(jnp.int32, (_POSE_LANES, _ROW), 1)
    s3 = ((c3 // 3) * 16 + 3 + (c3 % 3) * 4 == r3).astype(jnp.float32)

    xy = _hdot(poses, sxy)
    t24_ref[...] = _hdot(poses, s3)

    cols = []
    for m in range(_M):
        diff = xy - anchor_ref[m, :]
        cols.append(jnp.sum(diff * diff, axis=-1, keepdims=True))
    d2 = jnp.concatenate(cols, axis=-1)

    lane = lax.broadcasted_iota(jnp.int32, (_R, _M), 1)
    dmin = jnp.min(d2, axis=-1, keepdims=True)
    mode = jnp.min(jnp.where(d2 <= dmin, lane, _M), axis=-1)

    base = i * _R
    gidx = (base + lax.iota(jnp.int32, _R)) * _M + mode
    idx_ref[...] = gidx.reshape(1, 1, _R)

    onehot = (lane == mode[:, None]).astype(jnp.float32)
    for cls_ref, f_ref in ((cls0_ref, f0_ref), (cls1_ref, f1_ref)):
        pred = cls_ref[...]
        p = 1.0 / (1.0 + jnp.exp(-pred))
        pt = (1.0 - p) * onehot + p * (1.0 - onehot)
        fw = (_ALPHA * onehot + (1.0 - _ALPHA) * (1.0 - onehot)) * pt * pt
        bce = (jnp.maximum(pred, 0.0) - pred * onehot
               + jnp.log(1.0 + jnp.exp(-jnp.abs(pred))))
        f_ref[...] = jnp.full((1, 1, 128), jnp.sum(bce * fw), jnp.float32)


_tc_main = pl.pallas_call(
    _tc_main_body,
    grid=(_G,),
    in_specs=[
        pl.BlockSpec((_R, _POSE_LANES), lambda i: (i, 0)),
        pl.BlockSpec((_R, _M), lambda i: (i, 0)),
        pl.BlockSpec((_R, _M), lambda i: (i, 0)),
        pl.BlockSpec((_M, 2 * _FW), lambda i: (0, 0)),
    ],
    out_specs=[
        pl.BlockSpec((1, 1, _R), lambda i: (i, 0, 0)),
        pl.BlockSpec((_R, _ROW), lambda i: (i, 0)),
        pl.BlockSpec((1, 1, 128), lambda i: (i, 0, 0)),
        pl.BlockSpec((1, 1, 128), lambda i: (i, 0, 0)),
    ],
    out_shape=[
        jax.ShapeDtypeStruct((_G, 1, _R), jnp.int32),
        jax.ShapeDtypeStruct((_BT, _ROW), jnp.float32),
        jax.ShapeDtypeStruct((_G, 1, 128), jnp.float32),
        jax.ShapeDtypeStruct((_G, 1, 128), jnp.float32),
    ],
)


def _tc_reg_body(g0_ref, g1_ref, t_ref, o0_ref, o1_ref):
    t = t_ref[...]
    o0_ref[...] = jnp.full((1, 1, 128), jnp.sum(jnp.abs(g0_ref[...] - t)),
                           jnp.float32)
    o1_ref[...] = jnp.full((1, 1, 128), jnp.sum(jnp.abs(g1_ref[...] - t)),
                           jnp.float32)


_tc_reg = pl.pallas_call(
    _tc_reg_body,
    grid=(_GC,),
    in_specs=[
        pl.BlockSpec((_RC, _ROW), lambda i: (i, 0)),
        pl.BlockSpec((_RC, _ROW), lambda i: (i, 0)),
        pl.BlockSpec((_RC, _ROW), lambda i: (i, 0)),
    ],
    out_specs=[
        pl.BlockSpec((1, 1, 128), lambda i: (i, 0, 0)),
        pl.BlockSpec((1, 1, 128), lambda i: (i, 0, 0)),
    ],
    out_shape=[
        jax.ShapeDtypeStruct((_GC, 1, 128), jnp.float32),
        jax.ShapeDtypeStruct((_GC, 1, 128), jnp.float32),
    ],
)


def _sc_gather_body(idx_hbm, t0_hbm, t1_hbm, o0_hbm, o1_hbm,
                    idx_v, r0_v, r1_v, s0, s1):
    wid = lax.axis_index("s") * 2 + lax.axis_index("c")
    base = wid * _NPW
    pltpu.sync_copy(idx_hbm.at[wid], idx_v)
    cps = []
    for j in range(_NCH):
        cps.append(pltpu.async_copy(
            t0_hbm.at[idx_v.at[j]], r0_v.at[pl.ds(j * _CHUNK, _CHUNK)], s0))
        cps.append(pltpu.async_copy(
            t1_hbm.at[idx_v.at[j]], r1_v.at[pl.ds(j * _CHUNK, _CHUNK)], s1))
    for cp in cps:
        cp.wait()
    pltpu.sync_copy(r0_v, o0_hbm.at[pl.ds(base, _NPW)])
    pltpu.sync_copy(r1_v, o1_hbm.at[pl.ds(base, _NPW)])


@functools.cache
def _sc_gather():
    return pl.kernel(
        _sc_gather_body,
        mesh=plsc.VectorSubcoreMesh(core_axis_name="c", subcore_axis_name="s"),
        compiler_params=pltpu.CompilerParams(use_tc_tiling_on_sc=False),
        out_type=[jax.ShapeDtypeStruct((_BT, _ROW), jnp.float32)] * 2,
        scratch_types=[
            pltpu.VMEM((_NCH, _CHUNK), jnp.int32),
            pltpu.VMEM((_NPW, _ROW), jnp.float32),
            pltpu.VMEM((_NPW, _ROW), jnp.float32),
            pltpu.SemaphoreType.DMA,
            pltpu.SemaphoreType.DMA,
        ],
    )


def kernel(diff_traj_reg_0, diff_traj_cls_0, diff_traj_reg_1, diff_traj_cls_1,
           future_ego_n_to_ego_curr, anchor):
    poses2 = future_ego_n_to_ego_curr.reshape(_BT, _POSE_LANES)
    cls0 = diff_traj_cls_0.reshape(_BT, _M)
    cls1 = diff_traj_cls_1.reshape(_BT, _M)
    anc = anchor.reshape(_M, 2 * _FW)
    tab0 = diff_traj_reg_0.reshape(_BT * _M, _ROW)
    tab1 = diff_traj_reg_1.reshape(_BT * _M, _ROW)

    idx3, t24, f0p, f1p = _tc_main(poses2, cls0, cls1, anc)
    g0, g1 = _sc_gather()(idx3.reshape(_NW, _NCH, _CHUNK), tab0, tab1)
    r0p, r1p = _tc_reg(g0, g1, t24)

    n_cls = float(_BT * _M)
    n_reg = float(_BT * _ROW)
    cls0_l = jnp.sum(f0p[:, 0, 0]) / n_cls
    cls1_l = jnp.sum(f1p[:, 0, 0]) / n_cls
    reg0_l = jnp.sum(r0p[:, 0, 0]) / n_reg
    reg1_l = jnp.sum(r1p[:, 0, 0]) / n_reg
    return (cls0_l * _CLS_W + reg0_l * _REG_W
            + cls1_l * _CLS_W + reg1_l * _REG_W)


# speedup vs baseline: 1.0412x; 1.0412x over previous
import functools

import jax
import jax.numpy as jnp
from jax import lax
from jax.experimental import pallas as pl
from jax.experimental.pallas import tpu as pltpu
from jax.experimental.pallas import tpu_sc as plsc

_CLS_W = 10.0
_REG_W = 8.0
_GAMMA = 2.0
_ALPHA = 0.25

_B, _T, _M, _FW, _D = 256, 128, 20, 8, 3
_BT = _B * _T
_ROW = _FW * _D
_POSE_LANES = _FW * 16

_R = 1024
_G = _BT // _R
_RC = 2048
_GC = _BT // _RC

_NW = 32
_NPW = _BT // _NW
_CHUNK = 128
_NCH = _NPW // _CHUNK


def _hdot(a, b, dims=None):
    if dims is None:
        return lax.dot(a, b, precision=lax.Precision.HIGHEST)
    return lax.dot_general(a, b, (dims, ((), ())), precision=lax.Precision.HIGHEST)


def _tc_main_body(poses_ref, cls0_ref, cls1_ref, anchor_ref,
                  idx_ref, t24_ref, f0_ref, f1_ref):
    i = pl.program_id(0)
    poses = poses_ref[...]
    anchor = anchor_ref[0, 0]

    r = lax.broadcasted_iota(jnp.int32, (_POSE_LANES, 2 * _FW), 0)
    c = lax.broadcasted_iota(jnp.int32, (_POSE_LANES, 2 * _FW), 1)
    sxy = ((c // 2) * 16 + 3 + (c % 2) * 4 == r).astype(jnp.float32)
    r3 = lax.broadcasted_iota(jnp.int32, (_POSE_LANES, _ROW), 0)
    c3 = lax.broadcasted_iota(jnp.int32, (_POSE_LANES, _ROW), 1)
    s3 = ((c3 // 3) * 16 + 3 + (c3 % 3) * 4 == r3).astype(jnp.float32)

    xy = _hdot(poses, sxy)
    t24_ref[...] = _hdot(poses, s3)

    xy2 = jnp.sum(xy * xy, axis=-1, keepdims=True)
    a2 = jnp.sum(anchor * anchor, axis=-1)
    cross = _hdot(xy, anchor, dims=((1,), (1,)))
    d2 = (xy2 - 2.0 * cross) + a2[None, :]

    lane = lax.broadcasted_iota(jnp.int32, (_R, _M), 1)
    dmin = jnp.min(d2, axis=-1, keepdims=True)
    mode = jnp.min(jnp.where(d2 <= dmin, lane, _M), axis=-1)

    base = i * _R
    gidx = (base + lax.iota(jnp.int32, _R)) * _M + mode
    idx_ref[...] = gidx.reshape(1, _R // _CHUNK, _CHUNK)

    onehot = (lane == mode[:, None]).astype(jnp.float32)
    for cls_ref, f_ref in ((cls0_ref, f0_ref), (cls1_ref, f1_ref)):
        pred = cls_ref[...]
        e = jnp.exp(-jnp.abs(pred))
        p = jnp.where(pred >= 0.0, 1.0 / (1.0 + e), e / (1.0 + e))
        pt = (1.0 - p) * onehot + p * (1.0 - onehot)
        fw = (_ALPHA * onehot + (1.0 - _ALPHA) * (1.0 - onehot)) * pt * pt
        bce = (jnp.maximum(pred, 0.0) - pred * onehot + jnp.log(1.0 + e))
        f_ref[...] = jnp.full((1, 1, 128), jnp.sum(bce * fw), jnp.float32)


_tc_main = pl.pallas_call(
    _tc_main_body,
    grid=(_G,),
    in_specs=[
        pl.BlockSpec((_R, _POSE_LANES), lambda i: (i, 0)),
        pl.BlockSpec((_R, _M), lambda i: (i, 0)),
        pl.BlockSpec((_R, _M), lambda i: (i, 0)),
        pl.BlockSpec((1, 1, _M, 2 * _FW), lambda i: (0, 0, 0, 0)),
    ],
    out_specs=[
        pl.BlockSpec((1, _R // _CHUNK, _CHUNK), lambda i: (i, 0, 0)),
        pl.BlockSpec((_R, _ROW), lambda i: (i, 0)),
        pl.BlockSpec((1, 1, 128), lambda i: (i, 0, 0)),
        pl.BlockSpec((1, 1, 128), lambda i: (i, 0, 0)),
    ],
    out_shape=[
        jax.ShapeDtypeStruct((_NW, _NPW // _CHUNK, _CHUNK), jnp.int32),
        jax.ShapeDtypeStruct((_BT, _ROW), jnp.float32),
        jax.ShapeDtypeStruct((_G, 1, 128), jnp.float32),
        jax.ShapeDtypeStruct((_G, 1, 128), jnp.float32),
    ],
)


def _tc_reg_body(g0_ref, g1_ref, t_ref, o0_ref, o1_ref):
    t = t_ref[...]
    o0_ref[...] = jnp.full((1, 1, 128), jnp.sum(jnp.abs(g0_ref[...] - t)),
                           jnp.float32)
    o1_ref[...] = jnp.full((1, 1, 128), jnp.sum(jnp.abs(g1_ref[...] - t)),
                           jnp.float32)


_tc_reg = pl.pallas_call(
    _tc_reg_body,
    grid=(_GC,),
    in_specs=[
        pl.BlockSpec((_RC, _ROW), lambda i: (i, 0)),
        pl.BlockSpec((_RC, _ROW), lambda i: (i, 0)),
        pl.BlockSpec((_RC, _ROW), lambda i: (i, 0)),
    ],
    out_specs=[
        pl.BlockSpec((1, 1, 128), lambda i: (i, 0, 0)),
        pl.BlockSpec((1, 1, 128), lambda i: (i, 0, 0)),
    ],
    out_shape=[
        jax.ShapeDtypeStruct((_GC, 1, 128), jnp.float32),
        jax.ShapeDtypeStruct((_GC, 1, 128), jnp.float32),
    ],
)


def _sc_gather_body(idx_hbm, t0_hbm, t1_hbm, o0_hbm, o1_hbm,
                    idx_v, r0_v, r1_v, s0, s1):
    wid = lax.axis_index("s") * 2 + lax.axis_index("c")
    base = wid * _NPW
    pltpu.sync_copy(idx_hbm.at[wid], idx_v)
    cps = []
    for j in range(_NCH):
        cps.append(pltpu.async_copy(
            t0_hbm.at[idx_v.at[j]], r0_v.at[pl.ds(j * _CHUNK, _CHUNK)], s0))
        cps.append(pltpu.async_copy(
            t1_hbm.at[idx_v.at[j]], r1_v.at[pl.ds(j * _CHUNK, _CHUNK)], s1))
    for cp in cps:
        cp.wait()
    pltpu.sync_copy(r0_v, o0_hbm.at[pl.ds(base, _NPW)])
    pltpu.sync_copy(r1_v, o1_hbm.at[pl.ds(base, _NPW)])


@functools.cache
def _sc_gather():
    return pl.kernel(
        _sc_gather_body,
        mesh=plsc.VectorSubcoreMesh(core_axis_name="c", subcore_axis_name="s"),
        compiler_params=pltpu.CompilerParams(use_tc_tiling_on_sc=False),
        out_type=[jax.ShapeDtypeStruct((_BT, _ROW), jnp.float32)] * 2,
        scratch_types=[
            pltpu.VMEM((_NCH, _CHUNK), jnp.int32),
            pltpu.VMEM((_NPW, _ROW), jnp.float32),
            pltpu.VMEM((_NPW, _ROW), jnp.float32),
            pltpu.SemaphoreType.DMA,
            pltpu.SemaphoreType.DMA,
        ],
    )


def kernel(diff_traj_reg_0, diff_traj_cls_0, diff_traj_reg_1, diff_traj_cls_1,
           future_ego_n_to_ego_curr, anchor):
    poses2 = future_ego_n_to_ego_curr.reshape(_BT, _POSE_LANES)
    cls0 = diff_traj_cls_0.reshape(_BT, _M)
    cls1 = diff_traj_cls_1.reshape(_BT, _M)
    tab0 = diff_traj_reg_0.reshape(_BT * _M, _ROW)
    tab1 = diff_traj_reg_1.reshape(_BT * _M, _ROW)

    idx3, t24, f0p, f1p = _tc_main(poses2, cls0, cls1, anchor)
    g0, g1 = _sc_gather()(idx3, tab0, tab1)
    r0p, r1p = _tc_reg(g0, g1, t24)

    n_cls = float(_BT * _M)
    n_reg = float(_BT * _ROW)
    cls0_l = jnp.sum(f0p[:, 0, 0]) / n_cls
    cls1_l = jnp.sum(f1p[:, 0, 0]) / n_cls
    reg0_l = jnp.sum(r0p[:, 0, 0]) / n_reg
    reg1_l = jnp.sum(r1p[:, 0, 0]) / n_reg
    return (cls0_l * _CLS_W + reg0_l * _REG_W
            + cls1_l * _CLS_W + reg1_l * _REG_W)


# speedup vs baseline: 13.9838x; 13.4299x over previous
import functools

import jax
import jax.numpy as jnp
from jax import lax
from jax.experimental import pallas as pl
from jax.experimental.pallas import tpu as pltpu
from jax.experimental.pallas import tpu_sc as plsc

_CLS_W = 10.0
_REG_W = 8.0
_ALPHA = 0.25

_B, _T, _M, _FW, _D = 256, 128, 20, 8, 3
_BT = _B * _T
_ROW = _FW * _D
_NEL = _BT * _ROW

_BB = 8
_G = _B // _BB
_BC = 32
_GC = _B // _BC

_NW = 32
_NPW = _NEL // _NW
_CHUNK = 128
_NCH = _NPW // _CHUNK
_GRP = 24
_NGRP = _NCH // _GRP


def _hdot(a, b, dims=((1,), (0,))):
    return lax.dot_general(a, b, (dims, ((), ())),
                           precision=lax.Precision.HIGHEST)


def _tc_main_body(poses_ref, cls0_ref, cls1_ref, anc_ref,
                  idx_ref, t24_ref, f0_ref, f1_ref):
    i = pl.program_id(0)
    at = anc_ref[0, 0]
    a2 = jnp.sum(at * at, axis=0)

    jj = lax.broadcasted_iota(jnp.int32, (_ROW, 128), 0)
    rr = lax.broadcasted_iota(jnp.int32, (_ROW, 128), 1)
    s3 = ((jj % _FW) * 16 + 3 + (jj // _FW) * 4 == rr).astype(jnp.float32)
    kk = lax.broadcasted_iota(jnp.int32, (16, 128), 0)
    r2 = lax.broadcasted_iota(jnp.int32, (16, 128), 1)
    sxy = ((kk // 2) * 16 + 3 + (kk % 2) * 4 == r2).astype(jnp.float32)

    tt24 = lax.broadcasted_iota(jnp.int32, (_ROW, 128), 1)
    miota = lax.broadcasted_iota(jnp.int32, (_M, 128), 0)

    modes = []
    for b in range(_BB):
        pb = poses_ref[b]
        xy = _hdot(sxy, pb)
        t24 = _hdot(s3, pb)
        t24_ref[b] = t24

        xy2 = jnp.sum(xy * xy, axis=0, keepdims=True)
        cross = _hdot(at, xy, dims=((0,), (0,)))
        d2 = (xy2 - 2.0 * cross) + a2[:, None]
        dmin = jnp.min(d2, axis=0, keepdims=True)
        mode = jnp.min(jnp.where(d2 <= dmin, miota, _M), axis=0)
        modes.append(mode.reshape(1, 128))

        bglob = i * _BB + b
        idx_ref[b] = ((bglob * _M + mode[None, :]) * _ROW + jj) * 128 + tt24

    mode3 = jnp.concatenate(modes, axis=0)
    mio3 = lax.broadcasted_iota(jnp.int32, (_M, _BB, 128), 0)
    onehot = (mio3 == mode3[None, :, :]).astype(jnp.float32)
    for cls_ref, f_ref in ((cls0_ref, f0_ref), (cls1_ref, f1_ref)):
        pred = cls_ref[...]
        e = jnp.exp(-jnp.abs(pred))
        p = jnp.where(pred >= 0.0, 1.0 / (1.0 + e), e / (1.0 + e))
        pt = (1.0 - p) * onehot + p * (1.0 - onehot)
        fw = (_ALPHA * onehot + (1.0 - _ALPHA) * (1.0 - onehot)) * pt * pt
        bce = (jnp.maximum(pred, 0.0) - pred * onehot + jnp.log(1.0 + e))
        f_ref[...] = jnp.full((1, 1, 128), jnp.sum(bce * fw), jnp.float32)


_tc_main = pl.pallas_call(
    _tc_main_body,
    grid=(_G,),
    in_specs=[
        pl.BlockSpec((_BB, 128, 128), lambda i: (i, 0, 0)),
        pl.BlockSpec((_M, _BB, 128), lambda i: (0, i, 0)),
        pl.BlockSpec((_M, _BB, 128), lambda i: (0, i, 0)),
        pl.BlockSpec((1, 1, 16, _M), lambda i: (0, 0, 0, 0)),
    ],
    out_specs=[
        pl.BlockSpec((_BB, _ROW, 128), lambda i: (i, 0, 0)),
        pl.BlockSpec((_BB, _ROW, 128), lambda i: (i, 0, 0)),
        pl.BlockSpec((1, 1, 128), lambda i: (i, 0, 0)),
        pl.BlockSpec((1, 1, 128), lambda i: (i, 0, 0)),
    ],
    out_shape=[
        jax.ShapeDtypeStruct((_B, _ROW, 128), jnp.int32),
        jax.ShapeDtypeStruct((_B, _ROW, 128), jnp.float32),
        jax.ShapeDtypeStruct((_G, 1, 128), jnp.float32),
        jax.ShapeDtypeStruct((_G, 1, 128), jnp.float32),
    ],
)


def _tc_reg_body(g0_ref, g1_ref, t_ref, o0_ref, o1_ref):
    t = t_ref[...]
    o0_ref[...] = jnp.full((1, 1, 128), jnp.sum(jnp.abs(g0_ref[...] - t)),
                           jnp.float32)
    o1_ref[...] = jnp.full((1, 1, 128), jnp.sum(jnp.abs(g1_ref[...] - t)),
                           jnp.float32)


_tc_reg = pl.pallas_call(
    _tc_reg_body,
    grid=(_GC,),
    in_specs=[
        pl.BlockSpec((_BC, _ROW, 128), lambda i: (i, 0, 0)),
        pl.BlockSpec((_BC, _ROW, 128), lambda i: (i, 0, 0)),
        pl.BlockSpec((_BC, _ROW, 128), lambda i: (i, 0, 0)),
    ],
    out_specs=[
        pl.BlockSpec((1, 1, 128), lambda i: (i, 0, 0)),
        pl.BlockSpec((1, 1, 128), lambda i: (i, 0, 0)),
    ],
    out_shape=[
        jax.ShapeDtypeStruct((_GC, 1, 128), jnp.float32),
        jax.ShapeDtypeStruct((_GC, 1, 128), jnp.float32),
    ],
)


def _sc_gather_body(idx_hbm, t0_hbm, t1_hbm, o0_hbm, o1_hbm,
                    idx_v, v0, v1, s0, s1):
    wid = lax.axis_index("s") * 2 + lax.axis_index("c")
    base = wid * _NPW
    pltpu.sync_copy(idx_hbm.at[pl.ds(base, _NPW)], idx_v)

    def group(g, _):
        cps = []
        for k in range(_GRP):
            sl = pl.ds((g * _GRP + k) * _CHUNK, _CHUNK)
            cps.append(pltpu.async_copy(t0_hbm.at[idx_v.at[sl]], v0.at[sl], s0))
            cps.append(pltpu.async_copy(t1_hbm.at[idx_v.at[sl]], v1.at[sl], s1))
        for cp in cps:
            cp.wait()
        return ()

    lax.fori_loop(0, _NGRP, group, (), unroll=False)
    pltpu.sync_copy(v0, o0_hbm.at[pl.ds(base, _NPW)])
    pltpu.sync_copy(v1, o1_hbm.at[pl.ds(base, _NPW)])


@functools.cache
def _sc_gather():
    return pl.kernel(
        _sc_gather_body,
        mesh=plsc.VectorSubcoreMesh(core_axis_name="c", subcore_axis_name="s"),
        compiler_params=pltpu.CompilerParams(use_tc_tiling_on_sc=False),
        out_type=[jax.ShapeDtypeStruct((_NEL,), jnp.float32)] * 2,
        scratch_types=[
            pltpu.VMEM((_NPW,), jnp.int32),
            pltpu.VMEM((_NPW,), jnp.float32),
            pltpu.VMEM((_NPW,), jnp.float32),
            pltpu.SemaphoreType.DMA,
            pltpu.SemaphoreType.DMA,
        ],
    )


def kernel(diff_traj_reg_0, diff_traj_cls_0, diff_traj_reg_1, diff_traj_cls_1,
           future_ego_n_to_ego_curr, anchor):
    poses_f = jnp.transpose(future_ego_n_to_ego_curr,
                            (0, 2, 3, 4, 1)).reshape(_B, 128, 128)
    cls0_t = jnp.transpose(diff_traj_cls_0, (2, 0, 1))
    cls1_t = jnp.transpose(diff_traj_cls_1, (2, 0, 1))
    anc_t = jnp.transpose(anchor, (0, 1, 3, 2))
    tab0 = jnp.transpose(diff_traj_reg_0, (0, 2, 4, 3, 1)).reshape(-1)
    tab1 = jnp.transpose(diff_traj_reg_1, (0, 2, 4, 3, 1)).reshape(-1)

    idx3, t24, f0p, f1p = _tc_main(poses_f, cls0_t, cls1_t, anc_t)
    g0f, g1f = _sc_gather()(idx3.reshape(-1), tab0, tab1)
    r0p, r1p = _tc_reg(g0f.reshape(_B, _ROW, 128),
                       g1f.reshape(_B, _ROW, 128), t24)

    n_cls = float(_BT * _M)
    n_reg = float(_NEL)
    cls0_l = jnp.sum(f0p[:, 0, 0]) / n_cls
    cls1_l = jnp.sum(f1p[:, 0, 0]) / n_cls
    reg0_l = jnp.sum(r0p[:, 0, 0]) / n_reg
    reg1_l = jnp.sum(r1p[:, 0, 0]) / n_reg
    return (cls0_l * _CLS_W + reg0_l * _REG_W
            + cls1_l * _CLS_W + reg1_l * _REG_W)


# speedup vs baseline: 14.0119x; 1.0020x over previous
import functools

import jax
import jax.numpy as jnp
from jax import lax
from jax.experimental import pallas as pl
from jax.experimental.pallas import tpu as pltpu
from jax.experimental.pallas import tpu_sc as plsc

_CLS_W = 10.0
_REG_W = 8.0
_ALPHA = 0.25

_B, _T, _M, _FW, _D = 256, 128, 20, 8, 3
_BT = _B * _T
_ROW = _FW * _D
_NEL = _BT * _ROW

_BB = 8
_G = _B // _BB
_BC = 32
_GC = _B // _BC

_NW = 32
_NPW = _NEL // _NW
_CHUNK = 128
_NCH = _NPW // _CHUNK
_GRP = 24
_NGRP = _NCH // _GRP


def _hdot(a, b, dims=((1,), (0,))):
    return lax.dot_general(a, b, (dims, ((), ())),
                           precision=lax.Precision.HIGHEST)


def _tc_main_body(poses_ref, cls0_ref, cls1_ref, anc_ref,
                  idx_ref, t24_ref, f0_ref, f1_ref):
    i = pl.program_id(0)
    at = anc_ref[0, 0]
    a2 = jnp.sum(at * at, axis=0)

    jj = lax.broadcasted_iota(jnp.int32, (_ROW, 128), 0)
    rr = lax.broadcasted_iota(jnp.int32, (_ROW, 128), 1)
    s3 = ((jj % _FW) * 16 + 3 + (jj // _FW) * 4 == rr).astype(jnp.float32)
    kk = lax.broadcasted_iota(jnp.int32, (16, 128), 0)
    r2 = lax.broadcasted_iota(jnp.int32, (16, 128), 1)
    sxy = ((kk // 2) * 16 + 3 + (kk % 2) * 4 == r2).astype(jnp.float32)

    tt24 = lax.broadcasted_iota(jnp.int32, (_ROW, 128), 1)
    miota = lax.broadcasted_iota(jnp.int32, (_M, 128), 0)

    modes = []
    for b in range(_BB):
        pb = poses_ref[b]
        xy = _hdot(sxy, pb)
        t24 = _hdot(s3, pb)
        t24_ref[b] = t24

        xy2 = jnp.sum(xy * xy, axis=0, keepdims=True)
        cross = _hdot(at, xy, dims=((0,), (0,)))
        d2 = (xy2 - 2.0 * cross) + a2[:, None]
        dmin = jnp.min(d2, axis=0, keepdims=True)
        mode = jnp.min(jnp.where(d2 <= dmin, miota, _M), axis=0)
        modes.append(mode.reshape(1, 128))

        bglob = i * _BB + b
        idx_ref[b] = ((bglob * _M + mode[None, :]) * _ROW + jj) * 128 + tt24

    mode3 = jnp.concatenate(modes, axis=0)
    mio3 = lax.broadcasted_iota(jnp.int32, (_M, _BB, 128), 0)
    onehot = (mio3 == mode3[None, :, :]).astype(jnp.float32)
    for cls_ref, f_ref in ((cls0_ref, f0_ref), (cls1_ref, f1_ref)):
        pred = cls_ref[...]
        e = jnp.exp(-jnp.abs(pred))
        p = jnp.where(pred >= 0.0, 1.0 / (1.0 + e), e / (1.0 + e))
        pt = (1.0 - p) * onehot + p * (1.0 - onehot)
        fw = (_ALPHA * onehot + (1.0 - _ALPHA) * (1.0 - onehot)) * pt * pt
        bce = (jnp.maximum(pred, 0.0) - pred * onehot + jnp.log(1.0 + e))
        f_ref[...] = jnp.full((1, 1, 128), jnp.sum(bce * fw), jnp.float32)


_tc_main = pl.pallas_call(
    _tc_main_body,
    grid=(_G,),
    in_specs=[
        pl.BlockSpec((_BB, 128, 128), lambda i: (i, 0, 0)),
        pl.BlockSpec((_M, _BB, 128), lambda i: (0, i, 0)),
        pl.BlockSpec((_M, _BB, 128), lambda i: (0, i, 0)),
        pl.BlockSpec((1, 1, 16, _M), lambda i: (0, 0, 0, 0)),
    ],
    out_specs=[
        pl.BlockSpec((_BB, _ROW, 128), lambda i: (i, 0, 0)),
        pl.BlockSpec((_BB, _ROW, 128), lambda i: (i, 0, 0)),
        pl.BlockSpec((1, 1, 128), lambda i: (i, 0, 0)),
        pl.BlockSpec((1, 1, 128), lambda i: (i, 0, 0)),
    ],
    out_shape=[
        jax.ShapeDtypeStruct((_B, _ROW, 128), jnp.int32),
        jax.ShapeDtypeStruct((_B, _ROW, 128), jnp.float32),
        jax.ShapeDtypeStruct((_G, 1, 128), jnp.float32),
        jax.ShapeDtypeStruct((_G, 1, 128), jnp.float32),
    ],
)


def _tc_reg_body(g0_ref, g1_ref, t_ref, o0_ref, o1_ref):
    t = t_ref[...]
    o0_ref[...] = jnp.full((1, 1, 128), jnp.sum(jnp.abs(g0_ref[...] - t)),
                           jnp.float32)
    o1_ref[...] = jnp.full((1, 1, 128), jnp.sum(jnp.abs(g1_ref[...] - t)),
                           jnp.float32)


_tc_reg = pl.pallas_call(
    _tc_reg_body,
    grid=(_GC,),
    in_specs=[
        pl.BlockSpec((_BC, _ROW, 128), lambda i: (i, 0, 0)),
        pl.BlockSpec((_BC, _ROW, 128), lambda i: (i, 0, 0)),
        pl.BlockSpec((_BC, _ROW, 128), lambda i: (i, 0, 0)),
    ],
    out_specs=[
        pl.BlockSpec((1, 1, 128), lambda i: (i, 0, 0)),
        pl.BlockSpec((1, 1, 128), lambda i: (i, 0, 0)),
    ],
    out_shape=[
        jax.ShapeDtypeStruct((_GC, 1, 128), jnp.float32),
        jax.ShapeDtypeStruct((_GC, 1, 128), jnp.float32),
    ],
)


def _sc_gather_body(idx_hbm, t0_hbm, t1_hbm, o0_hbm, o1_hbm,
                    idx_v, v0, v1, s0, s1):
    wid = lax.axis_index("s") * 2 + lax.axis_index("c")
    base = wid * _NPW
    pltpu.sync_copy(idx_hbm.at[pl.ds(base, _NPW)], idx_v)

    def fire(g):
        for k in range(_GRP):
            sl = pl.ds((g * _GRP + k) * _CHUNK, _CHUNK)
            pltpu.async_copy(t0_hbm.at[idx_v.at[sl]], v0.at[sl], s0)
            pltpu.async_copy(t1_hbm.at[idx_v.at[sl]], v1.at[sl], s1)

    def drain(g):
        for k in range(_GRP):
            sl = pl.ds((g * _GRP + k) * _CHUNK, _CHUNK)
            pltpu.make_async_copy(t0_hbm.at[idx_v.at[sl]], v0.at[sl], s0).wait()
            pltpu.make_async_copy(t1_hbm.at[idx_v.at[sl]], v1.at[sl], s1).wait()

    fire(0)

    def group(g, _):
        fire(g)
        drain(g - 1)
        return ()

    lax.fori_loop(1, _NGRP, group, (), unroll=False)
    drain(_NGRP - 1)
    pltpu.sync_copy(v0, o0_hbm.at[pl.ds(base, _NPW)])
    pltpu.sync_copy(v1, o1_hbm.at[pl.ds(base, _NPW)])


@functools.cache
def _sc_gather():
    return pl.kernel(
        _sc_gather_body,
        mesh=plsc.VectorSubcoreMesh(core_axis_name="c", subcore_axis_name="s"),
        compiler_params=pltpu.CompilerParams(use_tc_tiling_on_sc=False),
        out_type=[jax.ShapeDtypeStruct((_NEL,), jnp.float32)] * 2,
        scratch_types=[
            pltpu.VMEM((_NPW,), jnp.int32),
            pltpu.VMEM((_NPW,), jnp.float32),
            pltpu.VMEM((_NPW,), jnp.float32),
            pltpu.SemaphoreType.DMA,
            pltpu.SemaphoreType.DMA,
        ],
    )


def kernel(diff_traj_reg_0, diff_traj_cls_0, diff_traj_reg_1, diff_traj_cls_1,
           future_ego_n_to_ego_curr, anchor):
    poses_f = jnp.transpose(future_ego_n_to_ego_curr,
                            (0, 2, 3, 4, 1)).reshape(_B, 128, 128)
    cls0_t = jnp.transpose(diff_traj_cls_0, (2, 0, 1))
    cls1_t = jnp.transpose(diff_traj_cls_1, (2, 0, 1))
    anc_t = jnp.transpose(anchor, (0, 1, 3, 2))
    tab0 = jnp.transpose(diff_traj_reg_0, (0, 2, 4, 3, 1)).reshape(-1)
    tab1 = jnp.transpose(diff_traj_reg_1, (0, 2, 4, 3, 1)).reshape(-1)

    idx3, t24, f0p, f1p = _tc_main(poses_f, cls0_t, cls1_t, anc_t)
    g0f, g1f = _sc_gather()(idx3.reshape(-1), tab0, tab1)
    r0p, r1p = _tc_reg(g0f.reshape(_B, _ROW, 128),
                       g1f.reshape(_B, _ROW, 128), t24)

    n_cls = float(_BT * _M)
    n_reg = float(_NEL)
    cls0_l = jnp.sum(f0p[:, 0, 0]) / n_cls
    cls1_l = jnp.sum(f1p[:, 0, 0]) / n_cls
    reg0_l = jnp.sum(r0p[:, 0, 0]) / n_reg
    reg1_l = jnp.sum(r1p[:, 0, 0]) / n_reg
    return (cls0_l * _CLS_W + reg0_l * _REG_W
            + cls1_l * _CLS_W + reg1_l * _REG_W)


# speedup vs baseline: 17.0236x; 1.2149x over previous
import functools

import jax
import jax.numpy as jnp
from jax import lax
from jax.experimental import pallas as pl
from jax.experimental.pallas import tpu as pltpu
from jax.experimental.pallas import tpu_sc as plsc

_CLS_W = 10.0
_REG_W = 8.0
_ALPHA = 0.25

_B, _T, _M, _FW, _D = 256, 128, 20, 8, 3
_BT = _B * _T
_ROW = _FW * _D
_NEL = _BT * _ROW

_BB = 8
_G = _B // _BB
_BC = 32
_GC = _B // _BC

_NW = 32
_NPW = _NEL // _NW
_CHUNK = 128
_NCH = _NPW // _CHUNK
_GRP = 24
_NGRP = _NCH // _GRP


def _hdot(a, b, dims=((1,), (0,))):
    return lax.dot_general(a, b, (dims, ((), ())),
                           precision=lax.Precision.HIGHEST)


def _tc_main_body(poses_ref, cls0_ref, cls1_ref, anc_ref,
                  idx_ref, t24_ref, f0_ref, f1_ref):
    i = pl.program_id(0)
    at = anc_ref[0, 0]
    a2 = jnp.sum(at * at, axis=0)

    jj = lax.broadcasted_iota(jnp.int32, (_ROW, 128), 0)
    rr = lax.broadcasted_iota(jnp.int32, (_ROW, 128), 1)
    s3 = ((jj % _FW) * 16 + 3 + (jj // _FW) * 4 == rr).astype(jnp.float32)
    kk = lax.broadcasted_iota(jnp.int32, (16, 128), 0)
    r2 = lax.broadcasted_iota(jnp.int32, (16, 128), 1)
    sxy = ((kk // 2) * 16 + 3 + (kk % 2) * 4 == r2).astype(jnp.float32)

    tt24 = lax.broadcasted_iota(jnp.int32, (_ROW, 128), 1)
    lanes = _BB * 128

    sel = jnp.concatenate([sxy, s3], axis=0)
    pb = jnp.concatenate([poses_ref[b] for b in range(_BB)], axis=1)
    y = _hdot(sel, pb)
    xy = y[:16, :]
    xy2 = jnp.sum(xy * xy, axis=0, keepdims=True)
    cross = _hdot(at, xy, dims=((0,), (0,)))
    d2 = (xy2 - 2.0 * cross) + a2[:, None]
    miota = lax.broadcasted_iota(jnp.int32, (_M, lanes), 0)
    dmin = jnp.min(d2, axis=0, keepdims=True)
    mode = jnp.min(jnp.where(d2 <= dmin, miota, _M), axis=0,
                   keepdims=True)

    for b in range(_BB):
        t24_ref[b] = y[16:, b * 128:(b + 1) * 128]
        mode_b = mode[:, b * 128:(b + 1) * 128]
        bglob = i * _BB + b
        idx_ref[b] = ((bglob * _M + mode_b) * _ROW + jj) * 128 + tt24

    mode3 = jnp.concatenate(
        [mode[:, b * 128:(b + 1) * 128] for b in range(_BB)], axis=0)
    mio3 = lax.broadcasted_iota(jnp.int32, (_M, _BB, 128), 0)
    onehot = (mio3 == mode3[None, :, :]).astype(jnp.float32)
    for cls_ref, f_ref in ((cls0_ref, f0_ref), (cls1_ref, f1_ref)):
        pred = cls_ref[...]
        e = jnp.exp(-jnp.abs(pred))
        p = jnp.where(pred >= 0.0, 1.0 / (1.0 + e), e / (1.0 + e))
        pt = (1.0 - p) * onehot + p * (1.0 - onehot)
        fw = (_ALPHA * onehot + (1.0 - _ALPHA) * (1.0 - onehot)) * pt * pt
        bce = (jnp.maximum(pred, 0.0) - pred * onehot + jnp.log(1.0 + e))
        f_ref[...] = jnp.full((1, 1, 128), jnp.sum(bce * fw), jnp.float32)


_tc_main = pl.pallas_call(
    _tc_main_body,
    grid=(_G,),
    in_specs=[
        pl.BlockSpec((_BB, 128, 128), lambda i: (i, 0, 0)),
        pl.BlockSpec((_M, _BB, 128), lambda i: (0, i, 0)),
        pl.BlockSpec((_M, _BB, 128), lambda i: (0, i, 0)),
        pl.BlockSpec((1, 1, 16, _M), lambda i: (0, 0, 0, 0)),
    ],
    out_specs=[
        pl.BlockSpec((_BB, _ROW, 128), lambda i: (i, 0, 0)),
        pl.BlockSpec((_BB, _ROW, 128), lambda i: (i, 0, 0)),
        pl.BlockSpec((1, 1, 128), lambda i: (i, 0, 0)),
        pl.BlockSpec((1, 1, 128), lambda i: (i, 0, 0)),
    ],
    out_shape=[
        jax.ShapeDtypeStruct((_B, _ROW, 128), jnp.int32),
        jax.ShapeDtypeStruct((_B, _ROW, 128), jnp.float32),
        jax.ShapeDtypeStruct((_G, 1, 128), jnp.float32),
        jax.ShapeDtypeStruct((_G, 1, 128), jnp.float32),
    ],
)


def _tc_reg_body(g0_ref, g1_ref, t_ref, o0_ref, o1_ref):
    t = t_ref[...]
    o0_ref[...] = jnp.full((1, 1, 128), jnp.sum(jnp.abs(g0_ref[...] - t)),
                           jnp.float32)
    o1_ref[...] = jnp.full((1, 1, 128), jnp.sum(jnp.abs(g1_ref[...] - t)),
                           jnp.float32)


_tc_reg = pl.pallas_call(
    _tc_reg_body,
    grid=(_GC,),
    in_specs=[
        pl.BlockSpec((_BC, _ROW, 128), lambda i: (i, 0, 0)),
        pl.BlockSpec((_BC, _ROW, 128), lambda i: (i, 0, 0)),
        pl.BlockSpec((_BC, _ROW, 128), lambda i: (i, 0, 0)),
    ],
    out_specs=[
        pl.BlockSpec((1, 1, 128), lambda i: (i, 0, 0)),
        pl.BlockSpec((1, 1, 128), lambda i: (i, 0, 0)),
    ],
    out_shape=[
        jax.ShapeDtypeStruct((_GC, 1, 128), jnp.float32),
        jax.ShapeDtypeStruct((_GC, 1, 128), jnp.float32),
    ],
)


def _sc_gather_body(idx_hbm, t0_hbm, t1_hbm, o0_hbm, o1_hbm,
                    idx_v, v0, v1, s0, s1):
    wid = lax.axis_index("s") * 2 + lax.axis_index("c")
    base = wid * _NPW
    pltpu.sync_copy(idx_hbm.at[pl.ds(base, _NPW)], idx_v)

    def fire(g):
        for k in range(_GRP):
            sl = pl.ds((g * _GRP + k) * _CHUNK, _CHUNK)
            pltpu.async_copy(t0_hbm.at[idx_v.at[sl]], v0.at[sl], s0)
            pltpu.async_copy(t1_hbm.at[idx_v.at[sl]], v1.at[sl], s1)

    def drain(g):
        for k in range(_GRP):
            sl = pl.ds((g * _GRP + k) * _CHUNK, _CHUNK)
            pltpu.make_async_copy(t0_hbm.at[idx_v.at[sl]], v0.at[sl], s0).wait()
            pltpu.make_async_copy(t1_hbm.at[idx_v.at[sl]], v1.at[sl], s1).wait()

    fire(0)

    def group(g, _):
        fire(g)
        drain(g - 1)
        return ()

    lax.fori_loop(1, _NGRP, group, (), unroll=False)
    drain(_NGRP - 1)
    pltpu.sync_copy(v0, o0_hbm.at[pl.ds(base, _NPW)])
    pltpu.sync_copy(v1, o1_hbm.at[pl.ds(base, _NPW)])


@functools.cache
def _sc_gather():
    return pl.kernel(
        _sc_gather_body,
        mesh=plsc.VectorSubcoreMesh(core_axis_name="c", subcore_axis_name="s"),
        compiler_params=pltpu.CompilerParams(use_tc_tiling_on_sc=False),
        out_type=[jax.ShapeDtypeStruct((_NEL,), jnp.float32)] * 2,
        scratch_types=[
            pltpu.VMEM((_NPW,), jnp.int32),
            pltpu.VMEM((_NPW,), jnp.float32),
            pltpu.VMEM((_NPW,), jnp.float32),
            pltpu.SemaphoreType.DMA,
            pltpu.SemaphoreType.DMA,
        ],
    )


def kernel(diff_traj_reg_0, diff_traj_cls_0, diff_traj_reg_1, diff_traj_cls_1,
           future_ego_n_to_ego_curr, anchor):
    poses_f = jnp.transpose(future_ego_n_to_ego_curr,
                            (0, 2, 3, 4, 1)).reshape(_B, 128, 128)
    cls0_t = jnp.transpose(diff_traj_cls_0, (2, 0, 1))
    cls1_t = jnp.transpose(diff_traj_cls_1, (2, 0, 1))
    anc_t = jnp.transpose(anchor, (0, 1, 3, 2))
    tab0 = jnp.transpose(diff_traj_reg_0, (0, 2, 4, 3, 1)).reshape(-1)
    tab1 = jnp.transpose(diff_traj_reg_1, (0, 2, 4, 3, 1)).reshape(-1)

    idx3, t24, f0p, f1p = _tc_main(poses_f, cls0_t, cls1_t, anc_t)
    g0f, g1f = _sc_gather()(idx3.reshape(-1), tab0, tab1)
    r0p, r1p = _tc_reg(g0f.reshape(_B, _ROW, 128),
                       g1f.reshape(_B, _ROW, 128), t24)

    n_cls = float(_BT * _M)
    n_reg = float(_NEL)
    cls0_l = jnp.sum(f0p[:, 0, 0]) / n_cls
    cls1_l = jnp.sum(f1p[:, 0, 0]) / n_cls
    reg0_l = jnp.sum(r0p[:, 0, 0]) / n_reg
    reg1_l = jnp.sum(r1p[:, 0, 0]) / n_reg
    return (cls0_l * _CLS_W + reg0_l * _REG_W
            + cls1_l * _CLS_W + reg1_l * _REG_W)


# speedup vs baseline: 17.8869x; 1.0507x over previous
import functools

import jax
import jax.numpy as jnp
from jax import lax
from jax.experimental import pallas as pl
from jax.experimental.pallas import tpu as pltpu
from jax.experimental.pallas import tpu_sc as plsc

_CLS_W = 10.0
_REG_W = 8.0
_ALPHA = 0.25

_B, _T, _M, _FW, _D = 256, 128, 20, 8, 3
_BT = _B * _T
_ROW = _FW * _D
_NEL = _BT * _ROW

_BB = 8
_G = _B // _BB
_BC = 32
_GC = _B // _BC

_NW = 32
_NPW = _NEL // _NW
_CHUNK = 128
_NCH = _NPW // _CHUNK
_GRP = 24
_NGRP = _NCH // _GRP


def _hdot(a, b, dims=((1,), (0,))):
    return lax.dot_general(a, b, (dims, ((), ())),
                           precision=lax.Precision.HIGHEST)


def _tc_main_body(poses_ref, anc_ref, idx_ref, t24_ref, mode_ref):
    i = pl.program_id(0)
    at = anc_ref[0, 0]
    a2 = jnp.sum(at * at, axis=0)

    jj = lax.broadcasted_iota(jnp.int32, (_ROW, 128), 0)
    rr = lax.broadcasted_iota(jnp.int32, (_ROW, 128), 1)
    s3 = ((jj % _FW) * 16 + 3 + (jj // _FW) * 4 == rr).astype(jnp.float32)
    kk = lax.broadcasted_iota(jnp.int32, (16, 128), 0)
    r2 = lax.broadcasted_iota(jnp.int32, (16, 128), 1)
    sxy = ((kk // 2) * 16 + 3 + (kk % 2) * 4 == r2).astype(jnp.float32)

    tt24 = lax.broadcasted_iota(jnp.int32, (_ROW, 128), 1)
    lanes = _BB * 128

    sel = jnp.concatenate([sxy, s3], axis=0)
    pb = jnp.concatenate([poses_ref[b] for b in range(_BB)], axis=1)
    y = _hdot(sel, pb)
    xy = y[:16, :]
    xy2 = jnp.sum(xy * xy, axis=0, keepdims=True)
    cross = _hdot(at, xy, dims=((0,), (0,)))
    d2 = (xy2 - 2.0 * cross) + a2[:, None]
    miota = lax.broadcasted_iota(jnp.int32, (_M, lanes), 0)
    dmin = jnp.min(d2, axis=0, keepdims=True)
    mode = jnp.min(jnp.where(d2 <= dmin, miota, _M), axis=0,
                   keepdims=True)

    for b in range(_BB):
        t24_ref[b] = y[16:, b * 128:(b + 1) * 128]
        mode_b = mode[:, b * 128:(b + 1) * 128]
        bglob = i * _BB + b
        idx_ref[b] = ((bglob * _M + mode_b) * _ROW + jj) * 128 + tt24
        mode_ref[b] = mode_b


_tc_main = pl.pallas_call(
    _tc_main_body,
    grid=(_G,),
    in_specs=[
        pl.BlockSpec((_BB, 128, 128), lambda i: (i, 0, 0)),
        pl.BlockSpec((1, 1, 16, _M), lambda i: (0, 0, 0, 0)),
    ],
    out_specs=[
        pl.BlockSpec((_BB, _ROW, 128), lambda i: (i, 0, 0)),
        pl.BlockSpec((_BB, _ROW, 128), lambda i: (i, 0, 0)),
        pl.BlockSpec((_BB, 1, 128), lambda i: (i, 0, 0)),
    ],
    out_shape=[
        jax.ShapeDtypeStruct((_B, _ROW, 128), jnp.int32),
        jax.ShapeDtypeStruct((_B, _ROW, 128), jnp.float32),
        jax.ShapeDtypeStruct((_B, 1, 128), jnp.int32),
    ],
)


def _tc_focal_body(cls0_ref, cls1_ref, mode_ref, f0_ref, f1_ref):
    mode3 = mode_ref[:, 0, :]
    mio3 = lax.broadcasted_iota(jnp.int32, (_M, _BB, 128), 0)
    onehot = (mio3 == mode3[None, :, :]).astype(jnp.float32)
    for cls_ref, f_ref in ((cls0_ref, f0_ref), (cls1_ref, f1_ref)):
        pred = cls_ref[...]
        e = jnp.exp(-jnp.abs(pred))
        p = jnp.where(pred >= 0.0, 1.0 / (1.0 + e), e / (1.0 + e))
        pt = (1.0 - p) * onehot + p * (1.0 - onehot)
        fw = (_ALPHA * onehot + (1.0 - _ALPHA) * (1.0 - onehot)) * pt * pt
        bce = (jnp.maximum(pred, 0.0) - pred * onehot + jnp.log(1.0 + e))
        f_ref[...] = jnp.full((1, 1, 128), jnp.sum(bce * fw), jnp.float32)


_tc_focal = pl.pallas_call(
    _tc_focal_body,
    grid=(_G,),
    in_specs=[
        pl.BlockSpec((_M, _BB, 128), lambda i: (0, i, 0)),
        pl.BlockSpec((_M, _BB, 128), lambda i: (0, i, 0)),
        pl.BlockSpec((_BB, 1, 128), lambda i: (i, 0, 0)),
    ],
    out_specs=[
        pl.BlockSpec((1, 1, 128), lambda i: (i, 0, 0)),
        pl.BlockSpec((1, 1, 128), lambda i: (i, 0, 0)),
    ],
    out_shape=[
        jax.ShapeDtypeStruct((_G, 1, 128), jnp.float32),
        jax.ShapeDtypeStruct((_G, 1, 128), jnp.float32),
    ],
)


def _tc_reg_body(g0_ref, g1_ref, t_ref, o0_ref, o1_ref):
    t = t_ref[...]
    o0_ref[...] = jnp.full((1, 1, 128), jnp.sum(jnp.abs(g0_ref[...] - t)),
                           jnp.float32)
    o1_ref[...] = jnp.full((1, 1, 128), jnp.sum(jnp.abs(g1_ref[...] - t)),
                           jnp.float32)


_tc_reg = pl.pallas_call(
    _tc_reg_body,
    grid=(_GC,),
    in_specs=[
        pl.BlockSpec((_BC, _ROW, 128), lambda i: (i, 0, 0)),
        pl.BlockSpec((_BC, _ROW, 128), lambda i: (i, 0, 0)),
        pl.BlockSpec((_BC, _ROW, 128), lambda i: (i, 0, 0)),
    ],
    out_specs=[
        pl.BlockSpec((1, 1, 128), lambda i: (i, 0, 0)),
        pl.BlockSpec((1, 1, 128), lambda i: (i, 0, 0)),
    ],
    out_shape=[
        jax.ShapeDtypeStruct((_GC, 1, 128), jnp.float32),
        jax.ShapeDtypeStruct((_GC, 1, 128), jnp.float32),
    ],
)


def _sc_gather_body(idx_hbm, t0_hbm, t1_hbm, o0_hbm, o1_hbm,
                    idx_v, v0, v1, s0, s1):
    wid = lax.axis_index("s") * 2 + lax.axis_index("c")
    base = wid * _NPW
    pltpu.sync_copy(idx_hbm.at[pl.ds(base, _NPW)], idx_v)

    def fire(g):
        for k in range(_GRP):
            sl = pl.ds((g * _GRP + k) * _CHUNK, _CHUNK)
            pltpu.async_copy(t0_hbm.at[idx_v.at[sl]], v0.at[sl], s0)
            pltpu.async_copy(t1_hbm.at[idx_v.at[sl]], v1.at[sl], s1)

    def drain(g):
        for k in range(_GRP):
            sl = pl.ds((g * _GRP + k) * _CHUNK, _CHUNK)
            pltpu.make_async_copy(t0_hbm.at[idx_v.at[sl]], v0.at[sl], s0).wait()
            pltpu.make_async_copy(t1_hbm.at[idx_v.at[sl]], v1.at[sl], s1).wait()

    fire(0)

    def group(g, _):
        fire(g)
        drain(g - 1)
        return ()

    lax.fori_loop(1, _NGRP, group, (), unroll=False)
    drain(_NGRP - 1)
    pltpu.sync_copy(v0, o0_hbm.at[pl.ds(base, _NPW)])
    pltpu.sync_copy(v1, o1_hbm.at[pl.ds(base, _NPW)])


@functools.cache
def _sc_gather():
    return pl.kernel(
        _sc_gather_body,
        mesh=plsc.VectorSubcoreMesh(core_axis_name="c", subcore_axis_name="s"),
        compiler_params=pltpu.CompilerParams(use_tc_tiling_on_sc=False),
        out_type=[jax.ShapeDtypeStruct((_NEL,), jnp.float32)] * 2,
        scratch_types=[
            pltpu.VMEM((_NPW,), jnp.int32),
            pltpu.VMEM((_NPW,), jnp.float32),
            pltpu.VMEM((_NPW,), jnp.float32),
            pltpu.SemaphoreType.DMA,
            pltpu.SemaphoreType.DMA,
        ],
    )


def kernel(diff_traj_reg_0, diff_traj_cls_0, diff_traj_reg_1, diff_traj_cls_1,
           future_ego_n_to_ego_curr, anchor):
    poses_f = jnp.transpose(future_ego_n_to_ego_curr,
                            (0, 2, 3, 4, 1)).reshape(_B, 128, 128)
    cls0_t = jnp.transpose(diff_traj_cls_0, (2, 0, 1))
    cls1_t = jnp.transpose(diff_traj_cls_1, (2, 0, 1))
    anc_t = jnp.transpose(anchor, (0, 1, 3, 2))
    tab0 = jnp.transpose(diff_traj_reg_0, (0, 2, 4, 3, 1)).reshape(-1)
    tab1 = jnp.transpose(diff_traj_reg_1, (0, 2, 4, 3, 1)).reshape(-1)

    idx3, t24, mode3 = _tc_main(poses_f, anc_t)
    g0f, g1f = _sc_gather()(idx3.reshape(-1), tab0, tab1)
    f0p, f1p = _tc_focal(cls0_t, cls1_t, mode3)
    r0p, r1p = _tc_reg(g0f.reshape(_B, _ROW, 128),
                       g1f.reshape(_B, _ROW, 128), t24)

    n_cls = float(_BT * _M)
    n_reg = float(_NEL)
    cls0_l = jnp.sum(f0p[:, 0, 0]) / n_cls
    cls1_l = jnp.sum(f1p[:, 0, 0]) / n_cls
    reg0_l = jnp.sum(r0p[:, 0, 0]) / n_reg
    reg1_l = jnp.sum(r1p[:, 0, 0]) / n_reg
    return (cls0_l * _CLS_W + reg0_l * _REG_W
            + cls1_l * _CLS_W + reg1_l * _REG_W)
